# speedup vs baseline: 1.5736x; 1.5736x over previous
import functools
import jax
import jax.numpy as jnp
from jax import lax
from jax.experimental import pallas as pl
from jax.experimental.pallas import tpu as pltpu
from jax.experimental.pallas import tpu_sc as plsc

D_MODEL = 2048
SEQ_LEN = 8192
NC, NS = 2, 16
NW = NC * NS
B_PER_W = SEQ_LEN // NW
R = 32
N_CHUNKS = B_PER_W // R

_mesh = plsc.VectorSubcoreMesh(core_axis_name="c", subcore_axis_name="s")


@functools.partial(
    pl.kernel,
    out_type=jax.ShapeDtypeStruct((SEQ_LEN, D_MODEL), jnp.float32),
    mesh=_mesh,
    scratch_types=[
        pltpu.VMEM((B_PER_W,), jnp.int32),
        pltpu.VMEM((R, D_MODEL), jnp.float32),
        pltpu.SemaphoreType.DMA,
    ],
)
def _gather_kernel(x_hbm, pe_hbm, out_hbm, idx_v, rows_v, sem):
    wid = lax.axis_index("s") * NC + lax.axis_index("c")
    base = pl.multiple_of(wid * B_PER_W, B_PER_W)
    pltpu.sync_copy(x_hbm.at[pl.ds(base, B_PER_W)], idx_v)

    def body(i, carry):
        off = pl.multiple_of(i * R, R)
        pltpu.async_copy(pe_hbm.at[idx_v.at[pl.ds(off, R)]], rows_v, sem).wait()
        pltpu.sync_copy(rows_v, out_hbm.at[pl.ds(base + off, R)])
        return carry

    lax.fori_loop(0, N_CHUNKS, body, 0)


def kernel(x, pe):
    return _gather_kernel(x, pe)


# speedup vs baseline: 1.6684x; 1.0603x over previous
import functools
import jax
import jax.numpy as jnp
from jax import lax
from jax.experimental import pallas as pl
from jax.experimental.pallas import tpu as pltpu
from jax.experimental.pallas import tpu_sc as plsc

D_MODEL = 2048
SEQ_LEN = 8192
NC, NS = 2, 16
NW = NC * NS
B_PER_W = SEQ_LEN // NW
R = 8
NBUF = 4
LEAD = 2
N_CHUNKS = B_PER_W // R
N_OUTER = N_CHUNKS // NBUF

_mesh = plsc.VectorSubcoreMesh(core_axis_name="c", subcore_axis_name="s")


@functools.partial(
    pl.kernel,
    out_type=jax.ShapeDtypeStruct((SEQ_LEN, D_MODEL), jnp.float32),
    mesh=_mesh,
    scratch_types=[
        pltpu.VMEM((B_PER_W,), jnp.int32),
        [pltpu.VMEM((R, D_MODEL), jnp.float32) for _ in range(NBUF)],
        [pltpu.SemaphoreType.DMA for _ in range(NBUF)],
        [pltpu.SemaphoreType.DMA for _ in range(NBUF)],
    ],
)
def _gather_kernel(x_hbm, pe_hbm, out_hbm, idx_v, rows, gsems, ssems):
    wid = lax.axis_index("s") * NC + lax.axis_index("c")
    base = pl.multiple_of(wid * B_PER_W, B_PER_W)
    pltpu.sync_copy(x_hbm.at[pl.ds(base, B_PER_W)], idx_v)

    def fire_gather(i, b):
        off = pl.multiple_of(i * R, R)
        pltpu.async_copy(pe_hbm.at[idx_v.at[pl.ds(off, R)]], rows[b], gsems[b])

    def wait_gather(b):
        pltpu.make_async_copy(
            pe_hbm.at[idx_v.at[pl.ds(0, R)]], rows[b], gsems[b]
        ).wait()

    def fire_store(i, b):
        off = pl.multiple_of(i * R, R)
        pltpu.async_copy(rows[b], out_hbm.at[pl.ds(base + off, R)], ssems[b])

    def wait_store(b):
        pltpu.make_async_copy(
            rows[b], out_hbm.at[pl.ds(base, R)], ssems[b]
        ).wait()

    def visit(i, b, do_swait, do_gfire):
        bn = (b + LEAD) % NBUF
        if do_swait:
            wait_store(bn)
        if do_gfire:
            fire_gather(i + LEAD, bn)
        wait_gather(b)
        fire_store(i, b)

    for b in range(LEAD):
        fire_gather(b, b)
    for b in range(NBUF):
        visit(b, b, do_swait=(b >= NBUF - LEAD), do_gfire=True)

    def outer(o, carry):
        for b in range(NBUF):
            visit(o * NBUF + b, b, True, True)
        return carry

    lax.fori_loop(1, N_OUTER - 1, outer, 0)

    for b in range(NBUF):
        i = (N_OUTER - 1) * NBUF + b
        visit(i, b, True, do_gfire=(i + LEAD < N_CHUNKS))
    for b in range(LEAD):
        wait_store((N_CHUNKS - LEAD + b) % NBUF)


def kernel(x, pe):
    return _gather_kernel(x, pe)


# speedup vs baseline: 1.6692x; 1.0005x over previous
import functools
import jax
import jax.numpy as jnp
from jax import lax
from jax.experimental import pallas as pl
from jax.experimental.pallas import tpu as pltpu
from jax.experimental.pallas import tpu_sc as plsc

D_MODEL = 2048
SEQ_LEN = 8192
NC, NS = 2, 16
NW = NC * NS
B_PER_W = SEQ_LEN // NW
R = 16
NBUF = 2
LEAD = 1
N_CHUNKS = B_PER_W // R
N_OUTER = N_CHUNKS // NBUF

_mesh = plsc.VectorSubcoreMesh(core_axis_name="c", subcore_axis_name="s")


@functools.partial(
    pl.kernel,
    out_type=jax.ShapeDtypeStruct((SEQ_LEN, D_MODEL), jnp.float32),
    mesh=_mesh,
    scratch_types=[
        pltpu.VMEM((B_PER_W,), jnp.int32),
        [pltpu.VMEM((R, D_MODEL), jnp.float32) for _ in range(NBUF)],
        [pltpu.SemaphoreType.DMA for _ in range(NBUF)],
        [pltpu.SemaphoreType.DMA for _ in range(NBUF)],
    ],
)
def _gather_kernel(x_hbm, pe_hbm, out_hbm, idx_v, rows, gsems, ssems):
    wid = lax.axis_index("s") * NC + lax.axis_index("c")
    base = pl.multiple_of(wid * B_PER_W, B_PER_W)
    pltpu.sync_copy(x_hbm.at[pl.ds(base, B_PER_W)], idx_v)

    def fire_gather(i, b):
        off = pl.multiple_of(i * R, R)
        pltpu.async_copy(pe_hbm.at[idx_v.at[pl.ds(off, R)]], rows[b], gsems[b])

    def wait_gather(b):
        pltpu.make_async_copy(
            pe_hbm.at[idx_v.at[pl.ds(0, R)]], rows[b], gsems[b]
        ).wait()

    def fire_store(i, b):
        off = pl.multiple_of(i * R, R)
        pltpu.async_copy(rows[b], out_hbm.at[pl.ds(base + off, R)], ssems[b])

    def wait_store(b):
        pltpu.make_async_copy(
            rows[b], out_hbm.at[pl.ds(base, R)], ssems[b]
        ).wait()

    def visit(i, b, do_swait, do_gfire):
        bn = (b + LEAD) % NBUF
        if do_swait:
            wait_store(bn)
        if do_gfire:
            fire_gather(i + LEAD, bn)
        wait_gather(b)
        fire_store(i, b)

    for b in range(LEAD):
        fire_gather(b, b)
    for b in range(NBUF):
        visit(b, b, do_swait=(b >= NBUF - LEAD), do_gfire=True)

    def outer(o, carry):
        for b in range(NBUF):
            visit(o * NBUF + b, b, True, True)
        return carry

    lax.fori_loop(1, N_OUTER - 1, outer, 0)

    for b in range(NBUF):
        i = (N_OUTER - 1) * NBUF + b
        visit(i, b, True, do_gfire=(i + LEAD < N_CHUNKS))
    for b in range(LEAD):
        wait_store((N_CHUNKS - LEAD + b) % NBUF)


def kernel(x, pe):
    return _gather_kernel(x, pe)
